# Initial kernel scaffold; baseline (speedup 1.0000x reference)
import jax
import jax.numpy as jnp
from jax.experimental import pallas as pl


def kernel(x, weight):
    raise NotImplementedError("write your pallas kernel here")


# speedup vs baseline: 1.1031x; 1.1031x over previous
import functools

import jax
import jax.numpy as jnp
from jax import lax
from jax.experimental import pallas as pl
from jax.experimental.pallas import tpu as pltpu
from jax.experimental.pallas import tpu_sc as plsc

_K = 128


@functools.partial(jax.jit, static_argnums=(2, 3, 4))
def _gather_sc(xr, weight, nw, nc, n_chunks):
    dim = weight.shape[1]
    mesh = plsc.VectorSubcoreMesh(core_axis_name="c", subcore_axis_name="s")

    @functools.partial(
        pl.kernel,
        out_type=jax.ShapeDtypeStruct((nw, n_chunks, _K, dim), jnp.float32),
        mesh=mesh,
        scratch_types=[
            pltpu.VMEM((n_chunks, _K), jnp.int32),
            pltpu.VMEM((_K, dim), jnp.float32),
            pltpu.SemaphoreType.DMA,
        ],
        compiler_params=pltpu.CompilerParams(use_tc_tiling_on_sc=False),
    )
    def k(x_hbm, w_hbm, out_hbm, idx_v, rows_v, sem):
        wid = lax.axis_index("s") * nc + lax.axis_index("c")
        pltpu.sync_copy(x_hbm.at[wid], idx_v)

        def body(j, carry):
            pltpu.async_copy(w_hbm.at[idx_v.at[j]], rows_v, sem).wait()
            pltpu.sync_copy(rows_v, out_hbm.at[wid, j])
            return carry

        lax.fori_loop(0, n_chunks, body, 0)

    return k(xr, weight)


def kernel(x, weight):
    b, f = x.shape
    dim = weight.shape[1]
    flat = b * f
    info = plsc.get_sparse_core_info()
    nc, ns = info.num_cores, info.num_subcores
    nw = nc * ns
    n_chunks = flat // (nw * _K)
    xr = x.reshape(nw, n_chunks, _K)
    out = _gather_sc(xr, weight, nw, nc, n_chunks)
    return out.reshape(b, f, dim)


# speedup vs baseline: 1.2149x; 1.1013x over previous
import functools

import jax
import jax.numpy as jnp
from jax import lax
from jax.experimental import pallas as pl
from jax.experimental.pallas import tpu as pltpu
from jax.experimental.pallas import tpu_sc as plsc

_K = 128
_NBUF = 4


@functools.partial(jax.jit, static_argnums=(2, 3, 4))
def _gather_sc(xr, weight, nw, nc, n_chunks):
    dim = weight.shape[1]
    mesh = plsc.VectorSubcoreMesh(core_axis_name="c", subcore_axis_name="s")
    n_outer = (n_chunks + _NBUF - 1) // _NBUF

    @functools.partial(
        pl.kernel,
        out_type=jax.ShapeDtypeStruct((nw, n_chunks, _K, dim), jnp.float32),
        mesh=mesh,
        scratch_types=[
            pltpu.VMEM((n_chunks, _K), jnp.int32),
            pltpu.VMEM((_NBUF, _K, dim), jnp.float32),
            pltpu.SemaphoreType.DMA((_NBUF,)),
            pltpu.SemaphoreType.DMA((_NBUF,)),
        ],
        compiler_params=pltpu.CompilerParams(use_tc_tiling_on_sc=False),
    )
    def k(x_hbm, w_hbm, out_hbm, idx_v, rows_v, gsem, ssem):
        wid = lax.axis_index("s") * nc + lax.axis_index("c")
        pltpu.sync_copy(x_hbm.at[wid], idx_v)

        def start_gather(c, b):
            pltpu.make_async_copy(
                w_hbm.at[idx_v.at[c]], rows_v.at[b], gsem.at[b]
            ).start()

        def wait_gather(c, b):
            pltpu.make_async_copy(
                w_hbm.at[idx_v.at[c]], rows_v.at[b], gsem.at[b]
            ).wait()

        def start_store(c, b):
            pltpu.make_async_copy(
                rows_v.at[b], out_hbm.at[wid, c], ssem.at[b]
            ).start()

        def wait_store(c, b):
            pltpu.make_async_copy(
                rows_v.at[b], out_hbm.at[wid, c], ssem.at[b]
            ).wait()

        for b in range(_NBUF):
            start_gather(b, b)

        def body(j, carry):
            for b in range(_NBUF):
                c = j * _NBUF + b

                @pl.when(c < n_chunks)
                def _():
                    wait_gather(c, b)
                    start_store(c, b)

                    @pl.when(c + _NBUF < n_chunks)
                    def _():
                        wait_store(c, b)
                        start_gather(c + _NBUF, b)

            return carry

        lax.fori_loop(0, n_outer, body, 0)
        for b in range(_NBUF):
            wait_store(0, b)

    return k(xr, weight)


def kernel(x, weight):
    b, f = x.shape
    dim = weight.shape[1]
    flat = b * f
    info = plsc.get_sparse_core_info()
    nc, ns = info.num_cores, info.num_subcores
    nw = nc * ns
    n_chunks = flat // (nw * _K)
    xr = x.reshape(nw, n_chunks, _K)
    out = _gather_sc(xr, weight, nw, nc, n_chunks)
    return out.reshape(b, f, dim)
